# baseline (speedup 1.0000x reference)
import jax
import jax.numpy as jnp
from jax.experimental import pallas as pl
from jax.experimental.pallas import tpu as pltpu

_CHUNK = 128


def kernel(x, E_relative_position):
    batch, seq, _ = x.shape
    attrs, edim = E_relative_position.shape
    reps = batch * seq // attrs
    n_dma = reps // _CHUNK

    def body(tab_ref, out_hbm, buf0, buf1, sem):
        t = jnp.broadcast_to(tab_ref[...][None], (_CHUNK, attrs, edim))
        buf0[...] = t
        buf1[...] = t
        bufs = (buf0, buf1)
        copies = [
            pltpu.make_async_copy(
                bufs[k % 2], out_hbm.at[pl.ds(k * _CHUNK, _CHUNK)], sem
            )
            for k in range(n_dma)
        ]
        for c in copies:
            c.start()
        for c in copies:
            c.wait()

    out = pl.pallas_call(
        body,
        in_specs=[pl.BlockSpec(memory_space=pltpu.MemorySpace.VMEM)],
        out_specs=pl.BlockSpec(memory_space=pltpu.MemorySpace.HBM),
        out_shape=jax.ShapeDtypeStruct((reps, attrs, edim), jnp.float32),
        scratch_shapes=[
            pltpu.VMEM((_CHUNK, attrs, edim), jnp.float32),
            pltpu.VMEM((_CHUNK, attrs, edim), jnp.float32),
            pltpu.SemaphoreType.DMA,
        ],
    )(E_relative_position)
    return out.reshape(batch, seq, edim)


# speedup vs baseline: 1.0040x; 1.0040x over previous
import jax
import jax.numpy as jnp
from jax.experimental import pallas as pl
from jax.experimental.pallas import tpu as pltpu

_CHUNK = 128


def kernel(x, E_relative_position):
    batch, seq, _ = x.shape
    attrs, edim = E_relative_position.shape
    reps = batch * seq // attrs
    n_dma = reps // _CHUNK

    def body(tab_ref, out_hbm, buf, sem):
        buf[...] = jnp.broadcast_to(tab_ref[...][None], (_CHUNK, attrs, edim))
        copies = [
            pltpu.make_async_copy(
                buf, out_hbm.at[pl.ds(k * _CHUNK, _CHUNK)], sem
            )
            for k in range(n_dma)
        ]
        for c in copies:
            c.start()
        for c in copies:
            c.wait()

    out = pl.pallas_call(
        body,
        in_specs=[pl.BlockSpec(memory_space=pltpu.MemorySpace.VMEM)],
        out_specs=pl.BlockSpec(memory_space=pltpu.MemorySpace.HBM),
        out_shape=jax.ShapeDtypeStruct((reps, attrs, edim), jnp.float32),
        scratch_shapes=[
            pltpu.VMEM((_CHUNK, attrs, edim), jnp.float32),
            pltpu.SemaphoreType.DMA,
        ],
    )(E_relative_position)
    return out.reshape(batch, seq, edim)
